# baseline (speedup 1.0000x reference)
import jax
import jax.numpy as jnp
from jax.experimental import pallas as pl


def _score_kernel(u_ref, v_ref, o_ref):
    o_ref[...] = jnp.sum(u_ref[...] * v_ref[...], axis=-1)


def kernel(x_adm, x_drug, node_id_drug, edge_index_ad, edge_attr_ad, edge_index_da, edge_attr_da, labels_index, W_proj_adm, b_proj_adm, W_proj_drug, b_proj_drug, emb_drug, W_edge_ad, b_edge_ad, W_edge_da, b_edge_da, W_gnn, b_gnn, pe, gru_W_adm, gru_U_adm, gru_b_adm, gru_W_drug, gru_U_drug, gru_b_drug):
    T, N, D = x_adm.shape
    feats_adm = []
    feats_drug = []
    emb = jnp.take(emb_drug, node_id_drug, axis=0)
    for t in range(T):
        h_adm = x_adm[t] @ W_proj_adm + b_proj_adm
        h_drug = x_drug[t] @ W_proj_drug + b_proj_drug + emb
        ea_ad = edge_attr_ad[t] @ W_edge_ad + b_edge_ad
        ea_da = edge_attr_da[t] @ W_edge_da + b_edge_da
        src_ad = edge_index_ad[t, 0]
        dst_ad = edge_index_ad[t, 1]
        src_da = edge_index_da[t, 0]
        dst_da = edge_index_da[t, 1]
        for l in range(2):
            m_ad = jax.nn.relu(jnp.take(h_adm, src_ad, axis=0) + ea_ad)
            agg_drug = jax.ops.segment_sum(m_ad, dst_ad, num_segments=N)
            m_da = jax.nn.relu(jnp.take(h_drug, src_da, axis=0) + ea_da)
            agg_adm = jax.ops.segment_sum(m_da, dst_da, num_segments=N)
            h_drug_new = jax.nn.relu((agg_drug + h_drug) @ W_gnn[l, 0] + b_gnn[l, 0])
            h_adm_new = jax.nn.relu((agg_adm + h_adm) @ W_gnn[l, 1] + b_gnn[l, 1])
            h_adm, h_drug = h_adm_new, h_drug_new
        feats_adm.append(h_adm)
        feats_drug.append(h_drug)
    seq_adm = jnp.stack(feats_adm) + pe[:, None, :]
    seq_drug = jnp.stack(feats_drug) + pe[:, None, :]
    ori_adm = x_adm[0] @ W_proj_adm + b_proj_adm
    ori_drug = x_drug[0] @ W_proj_drug + b_proj_drug + emb

    def gru(seq, h, W, U, b):
        outs = []
        for t in range(seq.shape[0]):
            xt = seq[t]
            z = jax.nn.sigmoid(xt @ W[0] + h @ U[0] + b[0])
            r = jax.nn.sigmoid(xt @ W[1] + h @ U[1] + b[1])
            n = jnp.tanh(xt @ W[2] + (r * h) @ U[2] + b[2])
            h = (1.0 - z) * h + z * n
            outs.append(h)
        return jnp.stack(outs)

    dec_adm = gru(seq_adm, ori_adm, gru_W_adm, gru_U_adm, gru_b_adm)
    dec_drug = gru(seq_drug, ori_drug, gru_W_drug, gru_U_drug, gru_b_drug)
    tix = jnp.arange(T)[:, None]
    u = dec_adm[tix, labels_index[:, 0]]
    v = dec_drug[tix, labels_index[:, 1]]
    L = labels_index.shape[-1]
    scores = pl.pallas_call(
        _score_kernel,
        out_shape=jax.ShapeDtypeStruct((T, L), jnp.float32),
    )(u, v)
    return scores


# speedup vs baseline: 1.3352x; 1.3352x over previous
import functools

import jax
import jax.numpy as jnp
from jax import lax
from jax.experimental import pallas as pl
from jax.experimental.pallas import tpu as pltpu
from jax.experimental.pallas import tpu_sc as plsc


def _score_kernel(u_ref, v_ref, o_ref):
    o_ref[...] = jnp.sum(u_ref[...] * v_ref[...], axis=-1)


N_TILES = 16


def _make_dual_segment_sum(E, NP, D):
    R = E // 128
    rows_per_tile = NP // N_TILES
    mesh = plsc.VectorSubcoreMesh(core_axis_name="c", subcore_axis_name="s")

    @functools.partial(
        pl.kernel,
        mesh=mesh,
        out_type=jax.ShapeDtypeStruct((2, NP, D), jnp.float32),
        scratch_types=[
            pltpu.VMEM((128,), jnp.int32),
            pltpu.VMEM((128, D), jnp.float32),
            pltpu.VMEM_SHARED((NP, D), jnp.float32),
        ],
    )
    def seg2(m_a_hbm, idx_a_hbm, m_b_hbm, idx_b_hbm, zeros_hbm, out_hbm,
             idx_v, rows_v, acc):
        core = lax.axis_index("c")
        sid = lax.axis_index("s")
        pltpu.sync_copy(zeros_hbm.at[pl.ds(sid * rows_per_tile, rows_per_tile), :],
                        acc.at[pl.ds(sid * rows_per_tile, rows_per_tile), :])
        plsc.subcore_barrier()

        start = (sid * R) // N_TILES
        end = ((sid + 1) * R) // N_TILES

        def run(m_hbm, idx_hbm):
            def body(i, _):
                pltpu.sync_copy(idx_hbm.at[pl.ds(i * 128, 128)], idx_v)
                pltpu.sync_copy(m_hbm.at[pl.ds(i * 128, 128), :], rows_v)
                pltpu.sync_copy(rows_v, acc.at[idx_v], add=True)
                return ()
            lax.fori_loop(start, end, body, ())

        @pl.when(core == 0)
        def _():
            run(m_a_hbm, idx_a_hbm)

        @pl.when(core == 1)
        def _():
            run(m_b_hbm, idx_b_hbm)

        plsc.subcore_barrier()
        pltpu.sync_copy(acc.at[pl.ds(sid * rows_per_tile, rows_per_tile), :],
                        out_hbm.at[core, pl.ds(sid * rows_per_tile, rows_per_tile), :])

    return seg2


def _dual_segment_sum(m_a, idx_a, m_b, idx_b, N):
    E, D = m_a.shape
    NP = 10240
    k = _make_dual_segment_sum(E, NP, D)
    zeros = jnp.zeros((NP, D), jnp.float32)
    out = k(m_a, idx_a, m_b, idx_b, zeros)
    return out[0, :N], out[1, :N]


def kernel(x_adm, x_drug, node_id_drug, edge_index_ad, edge_attr_ad, edge_index_da, edge_attr_da, labels_index, W_proj_adm, b_proj_adm, W_proj_drug, b_proj_drug, emb_drug, W_edge_ad, b_edge_ad, W_edge_da, b_edge_da, W_gnn, b_gnn, pe, gru_W_adm, gru_U_adm, gru_b_adm, gru_W_drug, gru_U_drug, gru_b_drug):
    T, N, D = x_adm.shape
    feats_adm = []
    feats_drug = []
    emb = jnp.take(emb_drug, node_id_drug, axis=0)
    for t in range(T):
        h_adm = x_adm[t] @ W_proj_adm + b_proj_adm
        h_drug = x_drug[t] @ W_proj_drug + b_proj_drug + emb
        ea_ad = edge_attr_ad[t] @ W_edge_ad + b_edge_ad
        ea_da = edge_attr_da[t] @ W_edge_da + b_edge_da
        src_ad = edge_index_ad[t, 0]
        dst_ad = edge_index_ad[t, 1]
        src_da = edge_index_da[t, 0]
        dst_da = edge_index_da[t, 1]
        for l in range(2):
            m_ad = jax.nn.relu(jnp.take(h_adm, src_ad, axis=0) + ea_ad)
            m_da = jax.nn.relu(jnp.take(h_drug, src_da, axis=0) + ea_da)
            agg_drug, agg_adm = _dual_segment_sum(m_ad, dst_ad, m_da, dst_da, N)
            h_drug_new = jax.nn.relu((agg_drug + h_drug) @ W_gnn[l, 0] + b_gnn[l, 0])
            h_adm_new = jax.nn.relu((agg_adm + h_adm) @ W_gnn[l, 1] + b_gnn[l, 1])
            h_adm, h_drug = h_adm_new, h_drug_new
        feats_adm.append(h_adm)
        feats_drug.append(h_drug)
    seq_adm = jnp.stack(feats_adm) + pe[:, None, :]
    seq_drug = jnp.stack(feats_drug) + pe[:, None, :]
    ori_adm = x_adm[0] @ W_proj_adm + b_proj_adm
    ori_drug = x_drug[0] @ W_proj_drug + b_proj_drug + emb

    def gru(seq, h, W, U, b):
        outs = []
        for t in range(seq.shape[0]):
            xt = seq[t]
            z = jax.nn.sigmoid(xt @ W[0] + h @ U[0] + b[0])
            r = jax.nn.sigmoid(xt @ W[1] + h @ U[1] + b[1])
            n = jnp.tanh(xt @ W[2] + (r * h) @ U[2] + b[2])
            h = (1.0 - z) * h + z * n
            outs.append(h)
        return jnp.stack(outs)

    dec_adm = gru(seq_adm, ori_adm, gru_W_adm, gru_U_adm, gru_b_adm)
    dec_drug = gru(seq_drug, ori_drug, gru_W_drug, gru_U_drug, gru_b_drug)
    tix = jnp.arange(T)[:, None]
    u = dec_adm[tix, labels_index[:, 0]]
    v = dec_drug[tix, labels_index[:, 1]]
    L = labels_index.shape[-1]
    scores = pl.pallas_call(
        _score_kernel,
        out_shape=jax.ShapeDtypeStruct((T, L), jnp.float32),
    )(u, v)
    return scores


# speedup vs baseline: 3.3823x; 2.5332x over previous
import functools

import jax
import jax.numpy as jnp
from jax import lax
from jax.experimental import pallas as pl
from jax.experimental.pallas import tpu as pltpu
from jax.experimental.pallas import tpu_sc as plsc

NP = 10240
D = 128
N_TILES = 16
N_WORKERS = 32


def _score_kernel(u_ref, v_ref, o_ref):
    o_ref[...] = jnp.sum(u_ref[...] * v_ref[...], axis=-1)


def _m_kernel(g_ref, attr_ref, w_ref, b_ref, o_ref):
    ea = jnp.dot(attr_ref[...], w_ref[...], preferred_element_type=jnp.float32)
    o_ref[...] = jnp.maximum(g_ref[...] + ea + b_ref[...], 0.0)


def _edge_messages(g, attr, w, b):
    E = g.shape[0]
    BR = 2000
    return pl.pallas_call(
        _m_kernel,
        grid=(E // BR,),
        in_specs=[
            pl.BlockSpec((BR, D), lambda i: (i, 0)),
            pl.BlockSpec((BR, attr.shape[1]), lambda i: (i, 0)),
            pl.BlockSpec((attr.shape[1], D), lambda i: (0, 0)),
            pl.BlockSpec((1, D), lambda i: (0, 0)),
        ],
        out_specs=pl.BlockSpec((BR, D), lambda i: (i, 0)),
        out_shape=jax.ShapeDtypeStruct((E, D), jnp.float32),
    )(g, attr, w, b.reshape(1, D))


def _span(R, w, nw):
    return (R * w) // nw, (R * (w + 1)) // nw


def _make_gather(E):
    R = E // 128
    rpt = NP // N_TILES
    mesh = plsc.VectorSubcoreMesh(core_axis_name="c", subcore_axis_name="s")

    @functools.partial(
        pl.kernel,
        mesh=mesh,
        out_type=jax.ShapeDtypeStruct((E, D), jnp.float32),
        scratch_types=[
            pltpu.VMEM((2, 128), jnp.int32),
            pltpu.VMEM((256, D), jnp.float32),
            pltpu.VMEM_SHARED((NP, D), jnp.float32),
            pltpu.SemaphoreType.DMA((2,)),
            pltpu.SemaphoreType.DMA((2,)),
        ],
    )
    def gather(tab_hbm, idx_hbm, out_hbm, idx_v, rows_v, table, sem_i, sem_o):
        core = lax.axis_index("c")
        sid = lax.axis_index("s")
        w = core * N_TILES + sid
        pltpu.sync_copy(tab_hbm.at[pl.ds(sid * rpt, rpt), :],
                        table.at[pl.ds(sid * rpt, rpt), :])
        plsc.subcore_barrier()
        lo, hi = _span(R, w, N_WORKERS)

        def load_idx(i, b):
            pltpu.async_copy(idx_hbm.at[pl.ds(i * 128, 128)], idx_v.at[b],
                             sem_i.at[b])

        def wait_idx(i, b):
            pltpu.make_async_copy(idx_hbm.at[pl.ds(i * 128, 128)], idx_v.at[b],
                                  sem_i.at[b]).wait()

        def store_out(i, b):
            pltpu.async_copy(rows_v.at[pl.ds(b * 128, 128), :],
                             out_hbm.at[pl.ds(i * 128, 128), :], sem_o.at[b])

        def wait_out(i, b):
            pltpu.make_async_copy(rows_v.at[pl.ds(b * 128, 128), :],
                                  out_hbm.at[pl.ds(i * 128, 128), :],
                                  sem_o.at[b]).wait()

        def step(i, b, first):
            wait_idx(i, b)
            if not first:
                @pl.when(i - 2 >= lo)
                def _():
                    wait_out(i - 2, b)
            pltpu.sync_copy(table.at[idx_v.at[b]],
                            rows_v.at[pl.ds(b * 128, 128), :])
            store_out(i, b)

        @pl.when(lo < hi)
        def _():
            load_idx(lo, 0)

            @pl.when(lo + 1 < hi)
            def _():
                load_idx(lo + 1, 1)

            def body(jj, _):
                i0 = lo + jj * 2
                step(i0, 0, False)

                @pl.when(i0 + 2 < hi)
                def _():
                    load_idx(i0 + 2, 0)

                @pl.when(i0 + 1 < hi)
                def _():
                    step(i0 + 1, 1, False)

                    @pl.when(i0 + 3 < hi)
                    def _():
                        load_idx(i0 + 3, 1)
                return ()

            lax.fori_loop(0, (hi - lo + 1) // 2, body, ())
            last = hi - 1

            @pl.when(last >= lo)
            def _():
                wait_out(last, (last - lo) % 2)

            @pl.when(last - 1 >= lo)
            def _():
                wait_out(last - 1, (last - 1 - lo) % 2)

    return gather


def _make_segsum(E):
    R = E // 128
    rpt = NP // N_TILES
    mesh = plsc.VectorSubcoreMesh(core_axis_name="c", subcore_axis_name="s")

    @functools.partial(
        pl.kernel,
        mesh=mesh,
        out_type=jax.ShapeDtypeStruct((2, NP, D), jnp.float32),
        scratch_types=[
            pltpu.VMEM((2, 128), jnp.int32),
            pltpu.VMEM((256, D), jnp.float32),
            pltpu.VMEM_SHARED((NP, D), jnp.float32),
            pltpu.SemaphoreType.DMA((2,)),
            pltpu.SemaphoreType.DMA((2,)),
        ],
    )
    def segsum(m_hbm, idx_hbm, zeros_hbm, out_hbm, idx_v, rows_v, acc,
               sem_i, sem_m):
        core = lax.axis_index("c")
        sid = lax.axis_index("s")
        w = core * N_TILES + sid
        pltpu.sync_copy(zeros_hbm.at[pl.ds(sid * rpt, rpt), :],
                        acc.at[pl.ds(sid * rpt, rpt), :])
        plsc.subcore_barrier()
        lo, hi = _span(R, w, N_WORKERS)

        def load(i, b):
            pltpu.async_copy(idx_hbm.at[pl.ds(i * 128, 128)], idx_v.at[b],
                             sem_i.at[b])
            pltpu.async_copy(m_hbm.at[pl.ds(i * 128, 128), :],
                             rows_v.at[pl.ds(b * 128, 128), :], sem_m.at[b])

        def wait(i, b):
            pltpu.make_async_copy(idx_hbm.at[pl.ds(i * 128, 128)], idx_v.at[b],
                                  sem_i.at[b]).wait()
            pltpu.make_async_copy(m_hbm.at[pl.ds(i * 128, 128), :],
                                  rows_v.at[pl.ds(b * 128, 128), :],
                                  sem_m.at[b]).wait()

        def scat(b):
            pltpu.sync_copy(rows_v.at[pl.ds(b * 128, 128), :],
                            acc.at[idx_v.at[b]], add=True)

        @pl.when(lo < hi)
        def _():
            load(lo, 0)

            @pl.when(lo + 1 < hi)
            def _():
                load(lo + 1, 1)

            def body(jj, _):
                i0 = lo + jj * 2
                wait(i0, 0)

                @pl.when(i0 + 2 < hi)
                def _():
                    load(i0 + 2, 0)
                scat(0)

                @pl.when(i0 + 1 < hi)
                def _():
                    wait(i0 + 1, 1)

                    @pl.when(i0 + 3 < hi)
                    def _():
                        load(i0 + 3, 1)
                    scat(1)
                return ()

            lax.fori_loop(0, (hi - lo + 1) // 2, body, ())

        plsc.subcore_barrier()
        pltpu.sync_copy(acc.at[pl.ds(sid * rpt, rpt), :],
                        out_hbm.at[core, pl.ds(sid * rpt, rpt), :])

    return segsum


def kernel(x_adm, x_drug, node_id_drug, edge_index_ad, edge_attr_ad, edge_index_da, edge_attr_da, labels_index, W_proj_adm, b_proj_adm, W_proj_drug, b_proj_drug, emb_drug, W_edge_ad, b_edge_ad, W_edge_da, b_edge_da, W_gnn, b_gnn, pe, gru_W_adm, gru_U_adm, gru_b_adm, gru_W_drug, gru_U_drug, gru_b_drug):
    T, N, _ = x_adm.shape
    E = edge_index_ad.shape[2]
    gather = _make_gather(E)
    segsum = _make_segsum(E)
    zeros = jnp.zeros((NP, D), jnp.float32)
    pad = jnp.zeros((NP - N, D), jnp.float32)

    emb = jnp.take(emb_drug, node_id_drug, axis=0)
    feats_adm = []
    feats_drug = []
    for t in range(T):
        h_adm = jnp.concatenate([x_adm[t] @ W_proj_adm + b_proj_adm, pad], axis=0)
        h_drug = jnp.concatenate([x_drug[t] @ W_proj_drug + b_proj_drug + emb, pad], axis=0)
        src_ad = edge_index_ad[t, 0]
        dst_ad = edge_index_ad[t, 1]
        src_da = edge_index_da[t, 0]
        dst_da = edge_index_da[t, 1]
        for l in range(2):
            g_ad = gather(h_adm, src_ad)
            m_ad = _edge_messages(g_ad, edge_attr_ad[t], W_edge_ad, b_edge_ad)
            g_da = gather(h_drug, src_da)
            m_da = _edge_messages(g_da, edge_attr_da[t], W_edge_da, b_edge_da)
            p_ad = segsum(m_ad, dst_ad, zeros)
            agg_drug = p_ad[0] + p_ad[1]
            p_da = segsum(m_da, dst_da, zeros)
            agg_adm = p_da[0] + p_da[1]
            h_drug = jax.nn.relu((agg_drug + h_drug) @ W_gnn[l, 0] + b_gnn[l, 0])
            h_adm = jax.nn.relu((agg_adm + h_adm) @ W_gnn[l, 1] + b_gnn[l, 1])
        feats_adm.append(h_adm[:N])
        feats_drug.append(h_drug[:N])
    seq_adm = jnp.stack(feats_adm) + pe[:, None, :]
    seq_drug = jnp.stack(feats_drug) + pe[:, None, :]
    ori_adm = x_adm[0] @ W_proj_adm + b_proj_adm
    ori_drug = x_drug[0] @ W_proj_drug + b_proj_drug + emb

    def gru(seq, h, W, U, b):
        outs = []
        for t in range(seq.shape[0]):
            xt = seq[t]
            z = jax.nn.sigmoid(xt @ W[0] + h @ U[0] + b[0])
            r = jax.nn.sigmoid(xt @ W[1] + h @ U[1] + b[1])
            n = jnp.tanh(xt @ W[2] + (r * h) @ U[2] + b[2])
            h = (1.0 - z) * h + z * n
            outs.append(h)
        return jnp.stack(outs)

    dec_adm = gru(seq_adm, ori_adm, gru_W_adm, gru_U_adm, gru_b_adm)
    dec_drug = gru(seq_drug, ori_drug, gru_W_drug, gru_U_drug, gru_b_drug)
    tix = jnp.arange(T)[:, None]
    u = dec_adm[tix, labels_index[:, 0]]
    v = dec_drug[tix, labels_index[:, 1]]
    L = labels_index.shape[-1]
    scores = pl.pallas_call(
        _score_kernel,
        out_shape=jax.ShapeDtypeStruct((T, L), jnp.float32),
    )(u, v)
    return scores
